# Initial kernel scaffold; baseline (speedup 1.0000x reference)
import jax
import jax.numpy as jnp
from jax.experimental import pallas as pl


def kernel(feats, edge_index, W1, al1, ar1, b1, W2, al2, ar2, b2):
    raise NotImplementedError("write your pallas kernel here")


# speedup vs baseline: 7.5534x; 7.5534x over previous
import functools

import jax
import jax.numpy as jnp
from jax import lax
from jax.experimental import pallas as pl
from jax.experimental.pallas import tpu as pltpu
from jax.experimental.pallas import tpu_sc as plsc

N = 10000
E = 320000
D_IN = 128
H1, D1 = 4, 32
H2, D2 = 1, 128
HD = 128

NC = 2
NS = 16
FH = HD // NC
EPT = E // NS
C = 2000
NSUB = EPT // C
B = 80
NBLK = C // B
RPT = N // NS
DC = 16
EW = 8

_f32 = jnp.float32
_i32 = jnp.int32


@functools.lru_cache(maxsize=None)
def _edge_kernel(H):
  mesh = plsc.VectorSubcoreMesh(core_axis_name="c", subcore_axis_name="s")
  nh = max(H // NC, 1)
  cph = FH // nh

  @functools.partial(
      pl.kernel,
      out_type=(
          jax.ShapeDtypeStruct((NC, N, FH), _f32),
          jax.ShapeDtypeStruct((NC, N, DC), _f32),
      ),
      mesh=mesh,
      compiler_params=pltpu.CompilerParams(use_tc_tiling_on_sc=False,
                                           needs_layout_passes=False),
      scratch_types=(
          pltpu.VMEM((NBLK, B), _i32),
          pltpu.VMEM((NBLK, B), _i32),
          pltpu.VMEM((C, EW), _f32),
          pltpu.VMEM((C, EW), _f32),
          pltpu.VMEM((C, DC), _f32),
          pltpu.VMEM((B, FH), _f32),
          pltpu.VMEM((B, FH), _f32),
          pltpu.VMEM_SHARED((N, FH), _f32),
          pltpu.VMEM_SHARED((N, DC), _f32),
          pltpu.SemaphoreType.DMA,
      ),
  )
  def k(feat, elr, src2d, dst2d, zrow, zden, out_hbm, den_hbm,
        srcv, dstv, elg, erg, eev, fstage, ostage, out_sh, den_sh, sem):
    core = lax.axis_index("c")
    sub = lax.axis_index("s")
    r0 = sub * RPT

    zv = jnp.zeros((16,), _f32)

    def zee(i, carry):
      eev[i, pl.ds(0, DC)] = zv
      return carry
    lax.fori_loop(0, C, zee, 0)

    pltpu.sync_copy(zrow, out_sh.at[pl.ds(r0, RPT)])
    pltpu.sync_copy(zden, den_sh.at[pl.ds(r0, RPT)])
    plsc.subcore_barrier()

    iota = lax.iota(_i32, 16)

    def subchunk(c0, carry):
      rbase = sub * (EPT // B) + c0 * NBLK
      pltpu.sync_copy(src2d.at[pl.ds(rbase, NBLK)], srcv)
      pltpu.sync_copy(dst2d.at[pl.ds(rbase, NBLK)], dstv)

      def gat(kk, carry2):
        pltpu.async_copy(elr.at[srcv.at[kk]], elg.at[pl.ds(kk * B, B)],
                         sem).wait()
        pltpu.async_copy(elr.at[dstv.at[kk]], erg.at[pl.ds(kk * B, B)],
                         sem).wait()
        return carry2
      lax.fori_loop(0, NBLK, gat, 0)

      def pha(i, carry2):
        rows = i * 16 + iota
        for h in range(H):
          el = plsc.load_gather(elg, [rows, jnp.full((16,), h, _i32)])
          er = plsc.load_gather(erg, [rows, jnp.full((16,), H + h, _i32)])
          e = el + er
          e = jnp.maximum(e, e * _f32(0.2))
          plsc.store_scatter(eev, [rows, jnp.full((16,), h, _i32)],
                             jnp.exp(e))
        return carry2
      lax.fori_loop(0, C // 16, pha, 0)

      def dden(kk, carry2):
        pltpu.sync_copy(eev.at[pl.ds(kk * B, B)], den_sh.at[dstv.at[kk]],
                        add=True)
        return carry2
      lax.fori_loop(0, NBLK, dden, 0)

      def phb(kk, carry2):
        pltpu.async_copy(feat.at[core].at[srcv.at[kk]], fstage, sem).wait()

        def scale(j, carry3):
          rows = j * 16 + iota
          erow = kk * B + j * 16 + iota
          alphas = [
              plsc.load_gather(
                  eev,
                  [erow, jnp.full((16,), t, _i32) + core * (H // NC)])
              for t in range(nh)
          ]
          for c in range(FH):
            a = alphas[c // cph]
            col = jnp.full((16,), c, _i32)
            v = plsc.load_gather(fstage, [rows, col])
            plsc.store_scatter(ostage, [rows, col], v * a)
          return carry3
        lax.fori_loop(0, B // 16, scale, 0)
        pltpu.sync_copy(ostage, out_sh.at[dstv.at[kk]], add=True)
        return carry2
      lax.fori_loop(0, NBLK, phb, 0)
      return carry
    lax.fori_loop(0, NSUB, subchunk, 0)

    plsc.subcore_barrier()
    pltpu.sync_copy(out_sh.at[pl.ds(r0, RPT)],
                    out_hbm.at[core, pl.ds(r0, RPT)])
    pltpu.sync_copy(den_sh.at[pl.ds(r0, RPT)],
                    den_hbm.at[core, pl.ds(r0, RPT)])

  return k


BS = 80
GRID = N // BS


def _lin_body(x_ref, w_ref, we_ref, feat_ref, elr_ref):
  x = x_ref[...]
  y = jnp.dot(x, w_ref[...], preferred_element_type=_f32)
  feat_ref[0] = y[:, :FH]
  feat_ref[1] = y[:, FH:]
  elr_ref[...] = jnp.dot(x, we_ref[...], preferred_element_type=_f32)


def _tc_lin(x, w, welr, h):
  return pl.pallas_call(
      _lin_body,
      grid=(GRID,),
      in_specs=[
          pl.BlockSpec((BS, HD), lambda i: (i, 0)),
          pl.BlockSpec((HD, HD), lambda i: (0, 0)),
          pl.BlockSpec((HD, EW), lambda i: (0, 0)),
      ],
      out_specs=[
          pl.BlockSpec((NC, BS, FH), lambda i: (0, i, 0)),
          pl.BlockSpec((BS, EW), lambda i: (i, 0)),
      ],
      out_shape=[
          jax.ShapeDtypeStruct((NC, N, FH), _f32),
          jax.ShapeDtypeStruct((N, EW), _f32),
      ],
  )(x, w, welr)


def _mid_body(h, op_ref, dp_ref, b_ref, s_ref, w_ref, we_ref,
              h_ref, feat_ref, elr_ref):
  acc = jnp.concatenate([op_ref[0], op_ref[1]], axis=1)
  den = dp_ref[0][:, :h]
  rden = _f32(1.0) / jnp.maximum(den, _f32(1e-9))
  rdenf = jnp.dot(rden, s_ref[...], preferred_element_type=_f32,
                  precision=lax.Precision.HIGHEST)
  hh = jnp.maximum(acc * rdenf + b_ref[...], _f32(0.0))
  h_ref[...] = hh
  y = jnp.dot(hh, w_ref[...], preferred_element_type=_f32)
  feat_ref[0] = y[:, :FH]
  feat_ref[1] = y[:, FH:]
  elr_ref[...] = jnp.dot(hh, we_ref[...], preferred_element_type=_f32)


def _tc_mid(outp, denp, bf, s, w, welr, h, h_next):
  return pl.pallas_call(
      functools.partial(_mid_body, h),
      grid=(GRID,),
      in_specs=[
          pl.BlockSpec((NC, BS, FH), lambda i: (0, i, 0)),
          pl.BlockSpec((1, BS, DC), lambda i: (0, i, 0)),
          pl.BlockSpec((1, HD), lambda i: (0, 0)),
          pl.BlockSpec((h, HD), lambda i: (0, 0)),
          pl.BlockSpec((HD, HD), lambda i: (0, 0)),
          pl.BlockSpec((HD, EW), lambda i: (0, 0)),
      ],
      out_specs=[
          pl.BlockSpec((BS, HD), lambda i: (i, 0)),
          pl.BlockSpec((NC, BS, FH), lambda i: (0, i, 0)),
          pl.BlockSpec((BS, EW), lambda i: (i, 0)),
      ],
      out_shape=[
          jax.ShapeDtypeStruct((N, HD), _f32),
          jax.ShapeDtypeStruct((NC, N, FH), _f32),
          jax.ShapeDtypeStruct((N, EW), _f32),
      ],
  )(outp, denp, bf, s, w, welr)


def _fin_body(h, op_ref, dp_ref, b_ref, s_ref, out_ref):
  acc = jnp.concatenate([op_ref[0], op_ref[1]], axis=1)
  den = dp_ref[0][:, :h]
  rden = _f32(1.0) / jnp.maximum(den, _f32(1e-9))
  rdenf = jnp.dot(rden, s_ref[...], preferred_element_type=_f32,
                  precision=lax.Precision.HIGHEST)
  out_ref[...] = acc * rdenf + b_ref[...]


def _tc_fin(outp, denp, bf, s, h):
  return pl.pallas_call(
      functools.partial(_fin_body, h),
      grid=(GRID,),
      in_specs=[
          pl.BlockSpec((NC, BS, FH), lambda i: (0, i, 0)),
          pl.BlockSpec((1, BS, DC), lambda i: (0, i, 0)),
          pl.BlockSpec((1, HD), lambda i: (0, 0)),
          pl.BlockSpec((h, HD), lambda i: (0, 0)),
      ],
      out_specs=pl.BlockSpec((BS, HD), lambda i: (i, 0)),
      out_shape=jax.ShapeDtypeStruct((N, HD), _f32),
  )(outp, denp, bf, s)


def kernel(feats, edge_index, W1, al1, ar1, b1, W2, al2, ar2, b2):
  src2d = edge_index[0].reshape(E // B, B)
  dst2d = edge_index[1].reshape(E // B, B)

  w1r = W1.reshape(D_IN, H1, D1)
  welr1 = jnp.concatenate(
      [jnp.einsum("ihd,hd->ih", w1r, al1),
       jnp.einsum("ihd,hd->ih", w1r, ar1)], axis=1)
  welr2 = jnp.concatenate(
      [W2 @ al2.T, W2 @ ar2.T,
       jnp.zeros((HD, EW - 2 * H2), _f32)], axis=1)

  zrow = jnp.zeros((RPT, FH), _f32)
  zden = jnp.zeros((RPT, DC), _f32)
  s4 = jnp.repeat(jnp.eye(H1, dtype=_f32), D1, axis=1)
  s1 = jnp.ones((H2, HD), _f32)
  b1f = b1.reshape(1, H1 * D1)
  b2f = b2.reshape(1, H2 * D2)

  feat1, elr1 = _tc_lin(feats, W1, welr1, H1)
  out1, den1 = _edge_kernel(H1)(feat1, elr1, src2d, dst2d, zrow, zden)
  h1, feat2, elr2 = _tc_mid(out1, den1, b1f, s4, W2, welr2, H1, H2)
  out2, den2 = _edge_kernel(H2)(feat2, elr2, src2d, dst2d, zrow, zden)
  h2 = _tc_fin(out2, den2, b2f, s1, H2)
  return (h2, feats, h1, h2)


# speedup vs baseline: 8.4871x; 1.1236x over previous
import functools

import jax
import jax.numpy as jnp
from jax import lax
from jax.experimental import pallas as pl
from jax.experimental.pallas import tpu as pltpu
from jax.experimental.pallas import tpu_sc as plsc

N = 10000
E = 320000
D_IN = 128
H1, D1 = 4, 32
H2, D2 = 1, 128
HD = 128

NC = 2
NS = 16
FH = HD // NC
EPT = E // NS
C = 2000
NSUB = EPT // C
B = 80
NBLK = C // B
RPT = N // NS
DC = 8
EW = 8

_f32 = jnp.float32
_i32 = jnp.int32


@functools.lru_cache(maxsize=None)
def _edge_kernel(H):
  mesh = plsc.VectorSubcoreMesh(core_axis_name="c", subcore_axis_name="s")
  nh = max(H // NC, 1)
  cph = FH // nh

  @functools.partial(
      pl.kernel,
      out_type=(
          jax.ShapeDtypeStruct((NC, N, FH), _f32),
          jax.ShapeDtypeStruct((NC, N, DC), _f32),
      ),
      mesh=mesh,
      compiler_params=pltpu.CompilerParams(use_tc_tiling_on_sc=False,
                                           needs_layout_passes=False),
      scratch_types=(
          pltpu.VMEM((NBLK, B), _i32),
          pltpu.VMEM((NBLK, B), _i32),
          pltpu.VMEM((C, EW), _f32),
          pltpu.VMEM((C, EW), _f32),
          pltpu.VMEM((C, DC), _f32),
          pltpu.VMEM((2, B, FH), _f32),
          pltpu.VMEM((2, B, FH), _f32),
          pltpu.VMEM_SHARED((N, FH), _f32),
          pltpu.VMEM_SHARED((N, DC), _f32),
          pltpu.SemaphoreType.DMA,
          pltpu.SemaphoreType.DMA((2,)),
          pltpu.SemaphoreType.DMA((2,)),
      ),
  )
  def k(feat, elr, src2d, dst2d, zrow, zden, out_hbm, den_hbm,
        srcv, dstv, elg, erg, eev, fstage, ostage, out_sh, den_sh, sem,
        gsem, ssem):
    core = lax.axis_index("c")
    sub = lax.axis_index("s")
    r0 = sub * RPT

    zv = jnp.zeros((16,), _f32)
    ziota = lax.iota(_i32, 16)
    zcol = ziota % DC
    zrow_l = ziota // DC

    def zee(i, carry):
      plsc.store_scatter(eev, [i * 2 + zrow_l, zcol], zv)
      return carry
    lax.fori_loop(0, C // 2, zee, 0)

    pltpu.sync_copy(zrow, out_sh.at[pl.ds(r0, RPT)])
    pltpu.sync_copy(zden, den_sh.at[pl.ds(r0, RPT)])
    plsc.subcore_barrier()

    iota = lax.iota(_i32, 16)

    def subchunk(c0, carry):
      rbase = sub * (EPT // B) + c0 * NBLK
      pltpu.sync_copy(src2d.at[pl.ds(rbase, NBLK)], srcv)
      pltpu.sync_copy(dst2d.at[pl.ds(rbase, NBLK)], dstv)

      def gat(kk, carry2):
        pltpu.async_copy(elr.at[srcv.at[kk]], elg.at[pl.ds(kk * B, B)],
                         sem).wait()
        pltpu.async_copy(elr.at[dstv.at[kk]], erg.at[pl.ds(kk * B, B)],
                         sem).wait()
        return carry2
      lax.fori_loop(0, NBLK, gat, 0)

      def pha(i, carry2):
        rows = i * 16 + iota
        for h in range(H):
          el = plsc.load_gather(elg, [rows, jnp.full((16,), h, _i32)])
          er = plsc.load_gather(erg, [rows, jnp.full((16,), H + h, _i32)])
          e = el + er
          e = jnp.maximum(e, e * _f32(0.2))
          plsc.store_scatter(eev, [rows, jnp.full((16,), h, _i32)],
                             jnp.exp(e))
        return carry2
      lax.fori_loop(0, C // 16, pha, 0)

      def dden(kk, carry2):
        pltpu.sync_copy(eev.at[pl.ds(kk * B, B)], den_sh.at[dstv.at[kk]],
                        add=True)
        return carry2
      lax.fori_loop(0, NBLK, dden, 0)

      def g_start(kk, p):
        pltpu.async_copy(feat.at[core].at[srcv.at[kk]], fstage.at[p],
                         gsem.at[p])

      def g_wait(kk, p):
        pltpu.make_async_copy(feat.at[core].at[srcv.at[kk]], fstage.at[p],
                              gsem.at[p]).wait()

      def s_start(kk, p):
        pltpu.async_copy(ostage.at[p], out_sh.at[dstv.at[kk]], ssem.at[p],
                         add=True)

      def s_wait(kk, p):
        pltpu.make_async_copy(ostage.at[p], out_sh.at[dstv.at[kk]],
                              ssem.at[p]).wait()

      g_start(0, 0)

      def phb(kk, carry2):
        p = kk % 2

        @pl.when(kk + 1 < NBLK)
        def _():
          g_start(kk + 1, 1 - p)
        g_wait(kk, p)

        @pl.when(kk >= 2)
        def _():
          s_wait(kk - 2, p)

        pp = jnp.full((16,), 0, _i32) + p

        def scale(j, carry3):
          rows = j * 16 + iota
          erow = kk * B + j * 16 + iota
          alphas = [
              plsc.load_gather(
                  eev,
                  [erow, jnp.full((16,), t, _i32) + core * (H // NC)])
              for t in range(nh)
          ]
          for c in range(FH):
            a = alphas[c // cph]
            col = jnp.full((16,), c, _i32)
            v = plsc.load_gather(fstage, [pp, rows, col])
            plsc.store_scatter(ostage, [pp, rows, col], v * a)
          return carry3
        lax.fori_loop(0, B // 16, scale, 0)
        s_start(kk, p)
        return carry2
      lax.fori_loop(0, NBLK, phb, 0)
      s_wait(NBLK - 2, (NBLK - 2) % 2)
      s_wait(NBLK - 1, (NBLK - 1) % 2)
      return carry
    lax.fori_loop(0, NSUB, subchunk, 0)

    plsc.subcore_barrier()
    pltpu.sync_copy(out_sh.at[pl.ds(r0, RPT)],
                    out_hbm.at[core, pl.ds(r0, RPT)])
    pltpu.sync_copy(den_sh.at[pl.ds(r0, RPT)],
                    den_hbm.at[core, pl.ds(r0, RPT)])

  return k


BS = 80
GRID = N // BS


def _lin_body(x_ref, w_ref, we_ref, feat_ref, elr_ref):
  x = x_ref[...]
  y = jnp.dot(x, w_ref[...], preferred_element_type=_f32)
  feat_ref[0] = y[:, :FH]
  feat_ref[1] = y[:, FH:]
  elr_ref[...] = jnp.dot(x, we_ref[...], preferred_element_type=_f32)


def _tc_lin(x, w, welr, h):
  return pl.pallas_call(
      _lin_body,
      grid=(GRID,),
      in_specs=[
          pl.BlockSpec((BS, HD), lambda i: (i, 0)),
          pl.BlockSpec((HD, HD), lambda i: (0, 0)),
          pl.BlockSpec((HD, EW), lambda i: (0, 0)),
      ],
      out_specs=[
          pl.BlockSpec((NC, BS, FH), lambda i: (0, i, 0)),
          pl.BlockSpec((BS, EW), lambda i: (i, 0)),
      ],
      out_shape=[
          jax.ShapeDtypeStruct((NC, N, FH), _f32),
          jax.ShapeDtypeStruct((N, EW), _f32),
      ],
  )(x, w, welr)


def _mid_body(h, op_ref, dp_ref, b_ref, s_ref, w_ref, we_ref,
              h_ref, feat_ref, elr_ref):
  acc = jnp.concatenate([op_ref[0], op_ref[1]], axis=1)
  den = dp_ref[0][:, :h]
  rden = _f32(1.0) / jnp.maximum(den, _f32(1e-9))
  rdenf = jnp.dot(rden, s_ref[...], preferred_element_type=_f32,
                  precision=lax.Precision.HIGHEST)
  hh = jnp.maximum(acc * rdenf + b_ref[...], _f32(0.0))
  h_ref[...] = hh
  y = jnp.dot(hh, w_ref[...], preferred_element_type=_f32)
  feat_ref[0] = y[:, :FH]
  feat_ref[1] = y[:, FH:]
  elr_ref[...] = jnp.dot(hh, we_ref[...], preferred_element_type=_f32)


def _tc_mid(outp, denp, bf, s, w, welr, h, h_next):
  return pl.pallas_call(
      functools.partial(_mid_body, h),
      grid=(GRID,),
      in_specs=[
          pl.BlockSpec((NC, BS, FH), lambda i: (0, i, 0)),
          pl.BlockSpec((1, BS, DC), lambda i: (0, i, 0)),
          pl.BlockSpec((1, HD), lambda i: (0, 0)),
          pl.BlockSpec((h, HD), lambda i: (0, 0)),
          pl.BlockSpec((HD, HD), lambda i: (0, 0)),
          pl.BlockSpec((HD, EW), lambda i: (0, 0)),
      ],
      out_specs=[
          pl.BlockSpec((BS, HD), lambda i: (i, 0)),
          pl.BlockSpec((NC, BS, FH), lambda i: (0, i, 0)),
          pl.BlockSpec((BS, EW), lambda i: (i, 0)),
      ],
      out_shape=[
          jax.ShapeDtypeStruct((N, HD), _f32),
          jax.ShapeDtypeStruct((NC, N, FH), _f32),
          jax.ShapeDtypeStruct((N, EW), _f32),
      ],
  )(outp, denp, bf, s, w, welr)


def _fin_body(h, op_ref, dp_ref, b_ref, s_ref, out_ref):
  acc = jnp.concatenate([op_ref[0], op_ref[1]], axis=1)
  den = dp_ref[0][:, :h]
  rden = _f32(1.0) / jnp.maximum(den, _f32(1e-9))
  rdenf = jnp.dot(rden, s_ref[...], preferred_element_type=_f32,
                  precision=lax.Precision.HIGHEST)
  out_ref[...] = acc * rdenf + b_ref[...]


def _tc_fin(outp, denp, bf, s, h):
  return pl.pallas_call(
      functools.partial(_fin_body, h),
      grid=(GRID,),
      in_specs=[
          pl.BlockSpec((NC, BS, FH), lambda i: (0, i, 0)),
          pl.BlockSpec((1, BS, DC), lambda i: (0, i, 0)),
          pl.BlockSpec((1, HD), lambda i: (0, 0)),
          pl.BlockSpec((h, HD), lambda i: (0, 0)),
      ],
      out_specs=pl.BlockSpec((BS, HD), lambda i: (i, 0)),
      out_shape=jax.ShapeDtypeStruct((N, HD), _f32),
  )(outp, denp, bf, s)


def kernel(feats, edge_index, W1, al1, ar1, b1, W2, al2, ar2, b2):
  src2d = edge_index[0].reshape(E // B, B)
  dst2d = edge_index[1].reshape(E // B, B)

  w1r = W1.reshape(D_IN, H1, D1)
  welr1 = jnp.concatenate(
      [jnp.einsum("ihd,hd->ih", w1r, al1),
       jnp.einsum("ihd,hd->ih", w1r, ar1)], axis=1)
  welr2 = jnp.concatenate(
      [W2 @ al2.T, W2 @ ar2.T,
       jnp.zeros((HD, EW - 2 * H2), _f32)], axis=1)

  zrow = jnp.zeros((RPT, FH), _f32)
  zden = jnp.zeros((RPT, DC), _f32)
  s4 = jnp.repeat(jnp.eye(H1, dtype=_f32), D1, axis=1)
  s1 = jnp.ones((H2, HD), _f32)
  b1f = b1.reshape(1, H1 * D1)
  b2f = b2.reshape(1, H2 * D2)

  feat1, elr1 = _tc_lin(feats, W1, welr1, H1)
  out1, den1 = _edge_kernel(H1)(feat1, elr1, src2d, dst2d, zrow, zden)
  h1, feat2, elr2 = _tc_mid(out1, den1, b1f, s4, W2, welr2, H1, H2)
  out2, den2 = _edge_kernel(H2)(feat2, elr2, src2d, dst2d, zrow, zden)
  h2 = _tc_fin(out2, den2, b2f, s1, H2)
  return (h2, feats, h1, h2)


# speedup vs baseline: 9.8785x; 1.1639x over previous
import functools

import jax
import jax.numpy as jnp
from jax import lax
from jax.experimental import pallas as pl
from jax.experimental.pallas import tpu as pltpu
from jax.experimental.pallas import tpu_sc as plsc

N = 10000
E = 320000
D_IN = 128
H1, D1 = 4, 32
H2, D2 = 1, 128
HD = 128

NC = 2
NS = 16
FH = HD // NC
EPT = E // NS
C = 2000
NSUB = EPT // C
B = 80
NBLK = C // B
RPT = N // NS
DC = 8
EW = 8

_f32 = jnp.float32
_i32 = jnp.int32


@functools.lru_cache(maxsize=None)
def _edge_kernel(H):
  mesh = plsc.VectorSubcoreMesh(core_axis_name="c", subcore_axis_name="s")
  nh = max(H // NC, 1)
  cph = FH // nh

  @functools.partial(
      pl.kernel,
      out_type=(
          jax.ShapeDtypeStruct((NC, N, FH), _f32),
          jax.ShapeDtypeStruct((NC, N, DC), _f32),
      ),
      mesh=mesh,
      compiler_params=pltpu.CompilerParams(use_tc_tiling_on_sc=False,
                                           needs_layout_passes=False),
      scratch_types=(
          pltpu.VMEM((NBLK, B), _i32),
          pltpu.VMEM((NBLK, B), _i32),
          pltpu.VMEM((C, EW), _f32),
          pltpu.VMEM((C, EW), _f32),
          pltpu.VMEM((C, DC), _f32),
          pltpu.VMEM((2, B, FH), _f32),
          pltpu.VMEM((2, B, FH), _f32),
          pltpu.VMEM_SHARED((N, FH), _f32),
          pltpu.VMEM_SHARED((N, DC), _f32),
          pltpu.SemaphoreType.DMA,
          pltpu.SemaphoreType.DMA((2,)),
          pltpu.SemaphoreType.DMA((2,)),
      ),
  )
  def k(feat, elr, src2d, dst2d, zrow, zden, out_hbm, den_hbm,
        srcv, dstv, elg, erg, eev, fstage, ostage, out_sh, den_sh, sem,
        gsem, ssem):
    core = lax.axis_index("c")
    sub = lax.axis_index("s")
    r0 = sub * RPT

    zv = jnp.zeros((16,), _f32)
    ziota = lax.iota(_i32, 16)
    zcol = ziota % DC
    zrow_l = ziota // DC

    def zee(i, carry):
      plsc.store_scatter(eev, [i * 2 + zrow_l, zcol], zv)
      return carry
    lax.fori_loop(0, C // 2, zee, 0)

    pltpu.sync_copy(zrow, out_sh.at[pl.ds(r0, RPT)])
    pltpu.sync_copy(zden, den_sh.at[pl.ds(r0, RPT)])
    plsc.subcore_barrier()

    iota = lax.iota(_i32, 16)

    def subchunk(c0, carry):
      rbase = sub * (EPT // B) + c0 * NBLK
      pltpu.sync_copy(src2d.at[pl.ds(rbase, NBLK)], srcv)
      pltpu.sync_copy(dst2d.at[pl.ds(rbase, NBLK)], dstv)

      def gat(kk, carry2):
        pltpu.async_copy(elr.at[srcv.at[kk]], elg.at[pl.ds(kk * B, B)],
                         sem)
        pltpu.async_copy(elr.at[dstv.at[kk]], erg.at[pl.ds(kk * B, B)],
                         sem)
        return carry2
      lax.fori_loop(0, NBLK, gat, 0)

      def gatw(kk, carry2):
        pltpu.make_async_copy(elr.at[srcv.at[kk]], elg.at[pl.ds(kk * B, B)],
                              sem).wait()
        pltpu.make_async_copy(elr.at[dstv.at[kk]], erg.at[pl.ds(kk * B, B)],
                              sem).wait()
        return carry2
      lax.fori_loop(0, NBLK, gatw, 0)

      def pha(i, carry2):
        rows = i * 16 + iota
        for h in range(H):
          el = plsc.load_gather(elg, [rows, jnp.full((16,), h, _i32)])
          er = plsc.load_gather(erg, [rows, jnp.full((16,), H + h, _i32)])
          e = el + er
          e = jnp.maximum(e, e * _f32(0.2))
          plsc.store_scatter(eev, [rows, jnp.full((16,), h, _i32)],
                             jnp.exp(e))
        return carry2
      lax.fori_loop(0, C // 16, pha, 0)

      def dden(kk, carry2):
        pltpu.async_copy(eev.at[pl.ds(kk * B, B)], den_sh.at[dstv.at[kk]],
                         sem, add=True)
        return carry2
      lax.fori_loop(0, NBLK, dden, 0)

      def ddenw(kk, carry2):
        pltpu.make_async_copy(eev.at[pl.ds(kk * B, B)],
                              den_sh.at[dstv.at[kk]], sem).wait()
        return carry2
      lax.fori_loop(0, NBLK, ddenw, 0)

      def g_start(kk, p):
        pltpu.async_copy(feat.at[core].at[srcv.at[kk]], fstage.at[p],
                         gsem.at[p])

      def g_wait(kk, p):
        pltpu.make_async_copy(feat.at[core].at[srcv.at[kk]], fstage.at[p],
                              gsem.at[p]).wait()

      def s_start(kk, p):
        pltpu.async_copy(ostage.at[p], out_sh.at[dstv.at[kk]], ssem.at[p],
                         add=True)

      def s_wait(kk, p):
        pltpu.make_async_copy(ostage.at[p], out_sh.at[dstv.at[kk]],
                              ssem.at[p]).wait()

      g_start(0, 0)

      def phb(kk, carry2):
        p = kk % 2

        @pl.when(kk + 1 < NBLK)
        def _():
          g_start(kk + 1, 1 - p)
        g_wait(kk, p)

        @pl.when(kk >= 2)
        def _():
          s_wait(kk - 2, p)

        pp = jnp.full((16,), 0, _i32) + p

        def scale(j, carry3):
          rows = j * 16 + iota
          erow = kk * B + j * 16 + iota
          alphas = [
              plsc.load_gather(
                  eev,
                  [erow, jnp.full((16,), t, _i32) + core * (H // NC)])
              for t in range(nh)
          ]
          for c in range(FH):
            a = alphas[c // cph]
            col = jnp.full((16,), c, _i32)
            v = plsc.load_gather(fstage, [pp, rows, col])
            plsc.store_scatter(ostage, [pp, rows, col], v * a)
          return carry3
        lax.fori_loop(0, B // 16, scale, 0)
        s_start(kk, p)
        return carry2
      lax.fori_loop(0, NBLK, phb, 0)
      s_wait(NBLK - 2, (NBLK - 2) % 2)
      s_wait(NBLK - 1, (NBLK - 1) % 2)
      return carry
    lax.fori_loop(0, NSUB, subchunk, 0)

    plsc.subcore_barrier()
    pltpu.sync_copy(out_sh.at[pl.ds(r0, RPT)],
                    out_hbm.at[core, pl.ds(r0, RPT)])
    pltpu.sync_copy(den_sh.at[pl.ds(r0, RPT)],
                    den_hbm.at[core, pl.ds(r0, RPT)])

  return k


BS = 80
GRID = N // BS


def _lin_body(x_ref, w_ref, we_ref, feat_ref, elr_ref):
  x = x_ref[...]
  y = jnp.dot(x, w_ref[...], preferred_element_type=_f32)
  feat_ref[0] = y[:, :FH]
  feat_ref[1] = y[:, FH:]
  elr_ref[...] = jnp.dot(x, we_ref[...], preferred_element_type=_f32)


def _tc_lin(x, w, welr, h):
  return pl.pallas_call(
      _lin_body,
      grid=(GRID,),
      in_specs=[
          pl.BlockSpec((BS, HD), lambda i: (i, 0)),
          pl.BlockSpec((HD, HD), lambda i: (0, 0)),
          pl.BlockSpec((HD, EW), lambda i: (0, 0)),
      ],
      out_specs=[
          pl.BlockSpec((NC, BS, FH), lambda i: (0, i, 0)),
          pl.BlockSpec((BS, EW), lambda i: (i, 0)),
      ],
      out_shape=[
          jax.ShapeDtypeStruct((NC, N, FH), _f32),
          jax.ShapeDtypeStruct((N, EW), _f32),
      ],
  )(x, w, welr)


def _mid_body(h, op_ref, dp_ref, b_ref, s_ref, w_ref, we_ref,
              h_ref, feat_ref, elr_ref):
  acc = jnp.concatenate([op_ref[0], op_ref[1]], axis=1)
  den = dp_ref[0][:, :h]
  rden = _f32(1.0) / jnp.maximum(den, _f32(1e-9))
  rdenf = jnp.dot(rden, s_ref[...], preferred_element_type=_f32,
                  precision=lax.Precision.HIGHEST)
  hh = jnp.maximum(acc * rdenf + b_ref[...], _f32(0.0))
  h_ref[...] = hh
  y = jnp.dot(hh, w_ref[...], preferred_element_type=_f32)
  feat_ref[0] = y[:, :FH]
  feat_ref[1] = y[:, FH:]
  elr_ref[...] = jnp.dot(hh, we_ref[...], preferred_element_type=_f32)


def _tc_mid(outp, denp, bf, s, w, welr, h, h_next):
  return pl.pallas_call(
      functools.partial(_mid_body, h),
      grid=(GRID,),
      in_specs=[
          pl.BlockSpec((NC, BS, FH), lambda i: (0, i, 0)),
          pl.BlockSpec((1, BS, DC), lambda i: (0, i, 0)),
          pl.BlockSpec((1, HD), lambda i: (0, 0)),
          pl.BlockSpec((h, HD), lambda i: (0, 0)),
          pl.BlockSpec((HD, HD), lambda i: (0, 0)),
          pl.BlockSpec((HD, EW), lambda i: (0, 0)),
      ],
      out_specs=[
          pl.BlockSpec((BS, HD), lambda i: (i, 0)),
          pl.BlockSpec((NC, BS, FH), lambda i: (0, i, 0)),
          pl.BlockSpec((BS, EW), lambda i: (i, 0)),
      ],
      out_shape=[
          jax.ShapeDtypeStruct((N, HD), _f32),
          jax.ShapeDtypeStruct((NC, N, FH), _f32),
          jax.ShapeDtypeStruct((N, EW), _f32),
      ],
  )(outp, denp, bf, s, w, welr)


def _fin_body(h, op_ref, dp_ref, b_ref, s_ref, out_ref):
  acc = jnp.concatenate([op_ref[0], op_ref[1]], axis=1)
  den = dp_ref[0][:, :h]
  rden = _f32(1.0) / jnp.maximum(den, _f32(1e-9))
  rdenf = jnp.dot(rden, s_ref[...], preferred_element_type=_f32,
                  precision=lax.Precision.HIGHEST)
  out_ref[...] = acc * rdenf + b_ref[...]


def _tc_fin(outp, denp, bf, s, h):
  return pl.pallas_call(
      functools.partial(_fin_body, h),
      grid=(GRID,),
      in_specs=[
          pl.BlockSpec((NC, BS, FH), lambda i: (0, i, 0)),
          pl.BlockSpec((1, BS, DC), lambda i: (0, i, 0)),
          pl.BlockSpec((1, HD), lambda i: (0, 0)),
          pl.BlockSpec((h, HD), lambda i: (0, 0)),
      ],
      out_specs=pl.BlockSpec((BS, HD), lambda i: (i, 0)),
      out_shape=jax.ShapeDtypeStruct((N, HD), _f32),
  )(outp, denp, bf, s)


def kernel(feats, edge_index, W1, al1, ar1, b1, W2, al2, ar2, b2):
  src2d = edge_index[0].reshape(E // B, B)
  dst2d = edge_index[1].reshape(E // B, B)

  w1r = W1.reshape(D_IN, H1, D1)
  welr1 = jnp.concatenate(
      [jnp.einsum("ihd,hd->ih", w1r, al1),
       jnp.einsum("ihd,hd->ih", w1r, ar1)], axis=1)
  welr2 = jnp.concatenate(
      [W2 @ al2.T, W2 @ ar2.T,
       jnp.zeros((HD, EW - 2 * H2), _f32)], axis=1)

  zrow = jnp.zeros((RPT, FH), _f32)
  zden = jnp.zeros((RPT, DC), _f32)
  s4 = jnp.repeat(jnp.eye(H1, dtype=_f32), D1, axis=1)
  s1 = jnp.ones((H2, HD), _f32)
  b1f = b1.reshape(1, H1 * D1)
  b2f = b2.reshape(1, H2 * D2)

  feat1, elr1 = _tc_lin(feats, W1, welr1, H1)
  out1, den1 = _edge_kernel(H1)(feat1, elr1, src2d, dst2d, zrow, zden)
  h1, feat2, elr2 = _tc_mid(out1, den1, b1f, s4, W2, welr2, H1, H2)
  out2, den2 = _edge_kernel(H2)(feat2, elr2, src2d, dst2d, zrow, zden)
  h2 = _tc_fin(out2, den2, b2f, s1, H2)
  return (h2, feats, h1, h2)
